# Initial kernel scaffold; baseline (speedup 1.0000x reference)
import jax
import jax.numpy as jnp
from jax.experimental import pallas as pl


def kernel(x, emb, W, b):
    raise NotImplementedError("write your pallas kernel here")


# speedup vs baseline: 3.3285x; 3.3285x over previous
import functools

import jax
import jax.numpy as jnp
from jax import lax
from jax.experimental import pallas as pl
from jax.experimental.pallas import tpu as pltpu
from jax.experimental.pallas import tpu_sc as plsc

HIDDEN = 128
NC, NS = 2, 16
NW = NC * NS
GK = 4
GROUP = GK * 128


def _fuse_body(emb_ref, w_ref, b_ref, out_ref):
    out_ref[...] = lax.dot_general(
        emb_ref[...], w_ref[...],
        dimension_numbers=(((1,), (1,)), ((), ())),
        preferred_element_type=jnp.float32,
    ) + b_ref[...]


def _fuse_table(emb_pad, W, b2):
    return pl.pallas_call(
        _fuse_body,
        out_shape=jax.ShapeDtypeStruct((HIDDEN, HIDDEN), jnp.float32),
    )(emb_pad, W, b2)


def _make_gather(n_tokens):
    per_w = n_tokens // NW
    groups = per_w // GROUP
    idx_rows_per_w = per_w // 128
    mesh = plsc.VectorSubcoreMesh(core_axis_name="c", subcore_axis_name="s")

    @functools.partial(
        pl.kernel,
        mesh=mesh,
        out_type=jax.ShapeDtypeStruct((n_tokens, HIDDEN), jnp.float32),
        scratch_types=[
            pltpu.VMEM((GK, 128), jnp.int32),
            pltpu.VMEM((GROUP, HIDDEN), jnp.float32),
            pltpu.SemaphoreType.DMA,
        ],
    )
    def gather(table_hbm, idx_hbm, out_hbm, idx_v, rows_v, sem):
        wid = lax.axis_index("s") * NC + lax.axis_index("c")
        row_base = wid * per_w
        idx_base = wid * idx_rows_per_w

        def body(g, carry):
            pltpu.sync_copy(idx_hbm.at[pl.ds(idx_base + g * GK, GK)], idx_v)
            copies = [
                pltpu.async_copy(
                    table_hbm.at[idx_v.at[j]],
                    rows_v.at[pl.ds(j * 128, 128)],
                    sem,
                )
                for j in range(GK)
            ]
            for cp in copies:
                cp.wait()
            pltpu.sync_copy(
                rows_v, out_hbm.at[pl.ds(row_base + g * GROUP, GROUP)])
            return carry

        lax.fori_loop(0, groups, body, 0)

    return gather


def kernel(x, emb, W, b):
    B, L = x.shape
    n = B * L
    emb_pad = jnp.pad(emb, ((0, HIDDEN - emb.shape[0]), (0, 0)))
    fused = _fuse_table(emb_pad, W, b.reshape(1, HIDDEN))
    idx = x.reshape(n // 128, 128)
    out = _make_gather(n)(fused, idx)
    return out.reshape(B, L, HIDDEN)


# speedup vs baseline: 3.3771x; 1.0146x over previous
import functools

import jax
import jax.numpy as jnp
from jax import lax
from jax.experimental import pallas as pl
from jax.experimental.pallas import tpu as pltpu
from jax.experimental.pallas import tpu_sc as plsc

HIDDEN = 128
NC, NS = 2, 16
NW = NC * NS
GK = 2
GROUP = GK * 128
CHUNK = 8


def _fuse_body(emb_ref, w_ref, b_ref, out_ref):
    out_ref[...] = lax.dot_general(
        emb_ref[...], w_ref[...],
        dimension_numbers=(((1,), (1,)), ((), ())),
        preferred_element_type=jnp.float32,
    ) + b_ref[...]


def _fuse_table(emb_pad, W, b2):
    return pl.pallas_call(
        _fuse_body,
        out_shape=jax.ShapeDtypeStruct((HIDDEN, HIDDEN), jnp.float32),
    )(emb_pad, W, b2)


def _make_gather(n_tokens):
    per_w = n_tokens // NW
    groups = per_w // GROUP
    supers = groups // CHUNK
    idx_rows_per_w = per_w // 128
    mesh = plsc.VectorSubcoreMesh(core_axis_name="c", subcore_axis_name="s")

    @functools.partial(
        pl.kernel,
        mesh=mesh,
        out_type=jax.ShapeDtypeStruct((n_tokens, HIDDEN), jnp.float32),
        scratch_types=[
            pltpu.VMEM((CHUNK * GK, 128), jnp.int32),
            pltpu.VMEM((GROUP, HIDDEN), jnp.float32),
            pltpu.VMEM((GROUP, HIDDEN), jnp.float32),
            pltpu.SemaphoreType.DMA,
            pltpu.SemaphoreType.DMA,
            pltpu.SemaphoreType.DMA,
            pltpu.SemaphoreType.DMA,
        ],
    )
    def gather(table_hbm, idx_hbm, out_hbm, idx_c, rows0, rows1,
               gsem0, gsem1, ssem0, ssem1):
        wid = lax.axis_index("s") * NC + lax.axis_index("c")
        row_base = wid * per_w
        idx_base = wid * idx_rows_per_w
        rows = (rows0, rows1)
        gsems = (gsem0, gsem1)
        ssems = (ssem0, ssem1)

        def load_chunk(ci):
            pltpu.sync_copy(
                idx_hbm.at[pl.ds(idx_base + ci * (CHUNK * GK), CHUNK * GK)],
                idx_c)

        def fire_gather(slot, p):
            for j2 in range(GK):
                pltpu.async_copy(
                    table_hbm.at[idx_c.at[slot * GK + j2]],
                    rows[p].at[pl.ds(j2 * 128, 128)],
                    gsems[p])

        def wait_gather(p):
            for j2 in range(GK):
                pltpu.make_async_copy(
                    table_hbm.at[idx_c.at[j2]],
                    rows[p].at[pl.ds(j2 * 128, 128)],
                    gsems[p]).wait()

        def fire_scatter(g, p):
            pltpu.async_copy(
                rows[p], out_hbm.at[pl.ds(row_base + g * GROUP, GROUP)],
                ssems[p])

        def wait_scatter(q):
            pltpu.make_async_copy(
                rows[q], out_hbm.at[pl.ds(row_base, GROUP)], ssems[q]).wait()

        def super_body(si, first=False, last=False):
            for j in range(CHUNK):
                g = si * CHUNK + j
                p = j % 2
                q = 1 - p
                wait_gather(p)
                fire_scatter(g, p)
                if not (first and j == 0):
                    wait_scatter(q)
                if not (last and j == CHUNK - 1):
                    if j == CHUNK - 1:
                        load_chunk(si + 1)
                        fire_gather(0, q)
                    else:
                        fire_gather(j + 1, q)

        load_chunk(0)
        fire_gather(0, 0)
        super_body(0, first=True)
        lax.fori_loop(1, supers - 1,
                      lambda si, c: (super_body(si), c)[1], 0)
        super_body(supers - 1, last=True)
        wait_scatter((CHUNK - 1) % 2)

    return gather


def kernel(x, emb, W, b):
    B, L = x.shape
    n = B * L
    emb_pad = jnp.pad(emb, ((0, HIDDEN - emb.shape[0]), (0, 0)))
    fused = _fuse_table(emb_pad, W, b.reshape(1, HIDDEN))
    idx = x.reshape(n // 128, 128)
    out = _make_gather(n)(fused, idx)
    return out.reshape(B, L, HIDDEN)


# speedup vs baseline: 10.2094x; 3.0232x over previous
import functools

import jax
import jax.numpy as jnp
from jax import lax
from jax.experimental import pallas as pl
from jax.experimental.pallas import tpu as pltpu
from jax.experimental.pallas import tpu_sc as plsc

HIDDEN = 128
NC, NS = 2, 16
NW = NC * NS
GK = 2
GROUP = GK * 128
CHUNK = 8


def _fuse_body(emb_ref, w_ref, b_ref, out_ref):
    fused = lax.dot_general(
        emb_ref[...], w_ref[...],
        dimension_numbers=(((1,), (1,)), ((), ())),
        preferred_element_type=jnp.float32,
    ) + b_ref[...]
    out_ref[...] = jnp.broadcast_to(fused[None], (NW, HIDDEN, HIDDEN))


def _fuse_table(emb_pad, W, b2):
    return pl.pallas_call(
        _fuse_body,
        out_shape=jax.ShapeDtypeStruct((NW, HIDDEN, HIDDEN), jnp.float32),
    )(emb_pad, W, b2)


def _make_gather(n_tokens):
    per_w = n_tokens // NW
    groups = per_w // GROUP
    supers = groups // CHUNK
    idx_rows_per_w = per_w // 128
    mesh = plsc.VectorSubcoreMesh(core_axis_name="c", subcore_axis_name="s")

    @functools.partial(
        pl.kernel,
        mesh=mesh,
        out_type=jax.ShapeDtypeStruct((n_tokens, HIDDEN), jnp.float32),
        scratch_types=[
            pltpu.VMEM((CHUNK * GK, 128), jnp.int32),
            pltpu.VMEM((GROUP, HIDDEN), jnp.float32),
            pltpu.VMEM((GROUP, HIDDEN), jnp.float32),
            pltpu.SemaphoreType.DMA,
            pltpu.SemaphoreType.DMA,
            pltpu.SemaphoreType.DMA,
            pltpu.SemaphoreType.DMA,
        ],
    )
    def gather(table_hbm, idx_hbm, out_hbm, idx_c, rows0, rows1,
               gsem0, gsem1, ssem0, ssem1):
        wid = lax.axis_index("s") * NC + lax.axis_index("c")
        row_base = wid * per_w
        idx_base = wid * idx_rows_per_w
        tab_off = wid * HIDDEN
        rows = (rows0, rows1)
        gsems = (gsem0, gsem1)
        ssems = (ssem0, ssem1)

        def load_chunk(ci):
            pltpu.sync_copy(
                idx_hbm.at[pl.ds(idx_base + ci * (CHUNK * GK), CHUNK * GK)],
                idx_c)
            for r in range(CHUNK * GK):
                for c in range(HIDDEN // 16):
                    sl = (r, pl.ds(c * 16, 16))
                    idx_c[sl] = idx_c[sl] + tab_off

        def fire_gather(slot, p):
            for j2 in range(GK):
                pltpu.async_copy(
                    table_hbm.at[idx_c.at[slot * GK + j2]],
                    rows[p].at[pl.ds(j2 * 128, 128)],
                    gsems[p])

        def wait_gather(p):
            for j2 in range(GK):
                pltpu.make_async_copy(
                    table_hbm.at[idx_c.at[j2]],
                    rows[p].at[pl.ds(j2 * 128, 128)],
                    gsems[p]).wait()

        def fire_scatter(g, p):
            pltpu.async_copy(
                rows[p], out_hbm.at[pl.ds(row_base + g * GROUP, GROUP)],
                ssems[p])

        def wait_scatter(q):
            pltpu.make_async_copy(
                rows[q], out_hbm.at[pl.ds(row_base, GROUP)], ssems[q]).wait()

        def super_body(si, first=False, last=False):
            for j in range(CHUNK):
                g = si * CHUNK + j
                p = j % 2
                q = 1 - p
                wait_gather(p)
                fire_scatter(g, p)
                if not (first and j == 0):
                    wait_scatter(q)
                if not (last and j == CHUNK - 1):
                    if j == CHUNK - 1:
                        load_chunk(si + 1)
                        fire_gather(0, q)
                    else:
                        fire_gather(j + 1, q)

        load_chunk(0)
        fire_gather(0, 0)
        super_body(0, first=True)
        lax.fori_loop(1, supers - 1,
                      lambda si, c: (super_body(si), c)[1], 0)
        super_body(supers - 1, last=True)
        wait_scatter((CHUNK - 1) % 2)

    return gather


def kernel(x, emb, W, b):
    B, L = x.shape
    n = B * L
    emb_pad = jnp.pad(emb, ((0, HIDDEN - emb.shape[0]), (0, 0)))
    fused = _fuse_table(emb_pad, W, b.reshape(1, HIDDEN))
    idx = x.reshape(n // 128, 128)
    out = _make_gather(n)(fused.reshape(NW * HIDDEN, HIDDEN), idx)
    return out.reshape(B, L, HIDDEN)


# speedup vs baseline: 10.9270x; 1.0703x over previous
import functools

import jax
import jax.numpy as jnp
from jax import lax
from jax.experimental import pallas as pl
from jax.experimental.pallas import tpu as pltpu
from jax.experimental.pallas import tpu_sc as plsc

HIDDEN = 128
NC, NS = 2, 16
NW = NC * NS
GK = 2
GROUP = GK * 128
CHUNK = 8
REP = 4


def _fuse_body(emb_ref, w_ref, b_ref, out_ref):
    fused = lax.dot_general(
        emb_ref[...], w_ref[...],
        dimension_numbers=(((1,), (1,)), ((), ())),
        preferred_element_type=jnp.float32,
    ) + b_ref[...]
    out_ref[...] = jnp.broadcast_to(fused[None], (NW * REP, HIDDEN, HIDDEN))


def _fuse_table(emb_pad, W, b2):
    return pl.pallas_call(
        _fuse_body,
        out_shape=jax.ShapeDtypeStruct((NW * REP, HIDDEN, HIDDEN), jnp.float32),
    )(emb_pad, W, b2)


def _make_gather(n_tokens):
    per_w = n_tokens // NW
    groups = per_w // GROUP
    supers = groups // CHUNK
    idx_rows_per_w = per_w // 128
    mesh = plsc.VectorSubcoreMesh(core_axis_name="c", subcore_axis_name="s")

    @functools.partial(
        pl.kernel,
        mesh=mesh,
        out_type=jax.ShapeDtypeStruct((n_tokens, HIDDEN), jnp.float32),
        scratch_types=[
            pltpu.VMEM((CHUNK * GK, 128), jnp.int32),
            pltpu.VMEM((GROUP, HIDDEN), jnp.float32),
            pltpu.VMEM((GROUP, HIDDEN), jnp.float32),
            pltpu.SemaphoreType.DMA,
            pltpu.SemaphoreType.DMA,
            pltpu.SemaphoreType.DMA,
            pltpu.SemaphoreType.DMA,
        ],
    )
    def gather(table_hbm, idx_hbm, out_hbm, idx_c, rows0, rows1,
               gsem0, gsem1, ssem0, ssem1):
        wid = lax.axis_index("s") * NC + lax.axis_index("c")
        row_base = wid * per_w
        idx_base = wid * idx_rows_per_w
        tab_off = wid * (REP * HIDDEN)
        rows = (rows0, rows1)
        gsems = (gsem0, gsem1)
        ssems = (ssem0, ssem1)

        def load_chunk(ci):
            pltpu.sync_copy(
                idx_hbm.at[pl.ds(idx_base + ci * (CHUNK * GK), CHUNK * GK)],
                idx_c)
            for r in range(CHUNK * GK):
                off = tab_off + (r % REP) * HIDDEN
                for c in range(HIDDEN // 16):
                    sl = (r, pl.ds(c * 16, 16))
                    idx_c[sl] = idx_c[sl] + off

        def fire_gather(slot, p):
            for j2 in range(GK):
                pltpu.async_copy(
                    table_hbm.at[idx_c.at[slot * GK + j2]],
                    rows[p].at[pl.ds(j2 * 128, 128)],
                    gsems[p])

        def wait_gather(p):
            for j2 in range(GK):
                pltpu.make_async_copy(
                    table_hbm.at[idx_c.at[j2]],
                    rows[p].at[pl.ds(j2 * 128, 128)],
                    gsems[p]).wait()

        def fire_scatter(g, p):
            pltpu.async_copy(
                rows[p], out_hbm.at[pl.ds(row_base + g * GROUP, GROUP)],
                ssems[p])

        def wait_scatter(q):
            pltpu.make_async_copy(
                rows[q], out_hbm.at[pl.ds(row_base, GROUP)], ssems[q]).wait()

        def super_body(si, first=False, last=False):
            for j in range(CHUNK):
                g = si * CHUNK + j
                p = j % 2
                q = 1 - p
                wait_gather(p)
                fire_scatter(g, p)
                if not (first and j == 0):
                    wait_scatter(q)
                if not (last and j == CHUNK - 1):
                    if j == CHUNK - 1:
                        load_chunk(si + 1)
                        fire_gather(0, q)
                    else:
                        fire_gather(j + 1, q)

        load_chunk(0)
        fire_gather(0, 0)
        super_body(0, first=True)
        lax.fori_loop(1, supers - 1,
                      lambda si, c: (super_body(si), c)[1], 0)
        super_body(supers - 1, last=True)
        wait_scatter((CHUNK - 1) % 2)

    return gather


def kernel(x, emb, W, b):
    B, L = x.shape
    n = B * L
    emb_pad = jnp.pad(emb, ((0, HIDDEN - emb.shape[0]), (0, 0)))
    fused = _fuse_table(emb_pad, W, b.reshape(1, HIDDEN))
    idx = x.reshape(n // 128, 128)
    out = _make_gather(n)(fused.reshape(NW * REP * HIDDEN, HIDDEN), idx)
    return out.reshape(B, L, HIDDEN)


# speedup vs baseline: 11.5587x; 1.0578x over previous
import functools

import jax
import jax.numpy as jnp
from jax import lax
from jax.experimental import pallas as pl
from jax.experimental.pallas import tpu as pltpu
from jax.experimental.pallas import tpu_sc as plsc

HIDDEN = 128
NC, NS = 2, 16
NW = NC * NS
GK = 2
GROUP = GK * 128
CHUNK = 8
REP = 8


def _fuse_body(emb_ref, w_ref, b_ref, out_ref):
    fused = lax.dot_general(
        emb_ref[...], w_ref[...],
        dimension_numbers=(((1,), (1,)), ((), ())),
        preferred_element_type=jnp.float32,
    ) + b_ref[...]
    out_ref[...] = jnp.broadcast_to(fused[None], (NW * REP, HIDDEN, HIDDEN))


def _fuse_table(emb_pad, W, b2):
    return pl.pallas_call(
        _fuse_body,
        out_shape=jax.ShapeDtypeStruct((NW * REP, HIDDEN, HIDDEN), jnp.float32),
    )(emb_pad, W, b2)


def _make_gather(n_tokens):
    per_w = n_tokens // NW
    groups = per_w // GROUP
    supers = groups // CHUNK
    idx_rows_per_w = per_w // 128
    mesh = plsc.VectorSubcoreMesh(core_axis_name="c", subcore_axis_name="s")

    @functools.partial(
        pl.kernel,
        mesh=mesh,
        out_type=jax.ShapeDtypeStruct((n_tokens, HIDDEN), jnp.float32),
        scratch_types=[
            pltpu.VMEM((CHUNK * GK, 128), jnp.int32),
            pltpu.VMEM((GROUP, HIDDEN), jnp.float32),
            pltpu.VMEM((GROUP, HIDDEN), jnp.float32),
            pltpu.SemaphoreType.DMA,
            pltpu.SemaphoreType.DMA,
            pltpu.SemaphoreType.DMA,
            pltpu.SemaphoreType.DMA,
        ],
    )
    def gather(table_hbm, idx_hbm, out_hbm, idx_c, rows0, rows1,
               gsem0, gsem1, ssem0, ssem1):
        wid = lax.axis_index("s") * NC + lax.axis_index("c")
        row_base = wid * per_w
        idx_base = wid * idx_rows_per_w
        tab_off = wid * (REP * HIDDEN)
        rows = (rows0, rows1)
        gsems = (gsem0, gsem1)
        ssems = (ssem0, ssem1)

        def load_chunk(ci):
            pltpu.sync_copy(
                idx_hbm.at[pl.ds(idx_base + ci * (CHUNK * GK), CHUNK * GK)],
                idx_c)
            lane = lax.iota(jnp.int32, 16)
            for r in range(CHUNK * GK):
                off = tab_off + ((lane + r) % REP) * HIDDEN
                for c in range(HIDDEN // 16):
                    sl = (r, pl.ds(c * 16, 16))
                    idx_c[sl] = idx_c[sl] + off

        def fire_gather(slot, p):
            for j2 in range(GK):
                pltpu.async_copy(
                    table_hbm.at[idx_c.at[slot * GK + j2]],
                    rows[p].at[pl.ds(j2 * 128, 128)],
                    gsems[p])

        def wait_gather(p):
            for j2 in range(GK):
                pltpu.make_async_copy(
                    table_hbm.at[idx_c.at[j2]],
                    rows[p].at[pl.ds(j2 * 128, 128)],
                    gsems[p]).wait()

        def fire_scatter(g, p):
            pltpu.async_copy(
                rows[p], out_hbm.at[pl.ds(row_base + g * GROUP, GROUP)],
                ssems[p])

        def wait_scatter(q):
            pltpu.make_async_copy(
                rows[q], out_hbm.at[pl.ds(row_base, GROUP)], ssems[q]).wait()

        def super_body(si, first=False, last=False):
            for j in range(CHUNK):
                g = si * CHUNK + j
                p = j % 2
                q = 1 - p
                wait_gather(p)
                fire_scatter(g, p)
                if not (first and j == 0):
                    wait_scatter(q)
                if not (last and j == CHUNK - 1):
                    if j == CHUNK - 1:
                        load_chunk(si + 1)
                        fire_gather(0, q)
                    else:
                        fire_gather(j + 1, q)

        load_chunk(0)
        fire_gather(0, 0)
        super_body(0, first=True)
        lax.fori_loop(1, supers - 1,
                      lambda si, c: (super_body(si), c)[1], 0)
        super_body(supers - 1, last=True)
        wait_scatter((CHUNK - 1) % 2)

    return gather


def kernel(x, emb, W, b):
    B, L = x.shape
    n = B * L
    emb_pad = jnp.pad(emb, ((0, HIDDEN - emb.shape[0]), (0, 0)))
    fused = _fuse_table(emb_pad, W, b.reshape(1, HIDDEN))
    idx = x.reshape(n // 128, 128)
    out = _make_gather(n)(fused.reshape(NW * REP * HIDDEN, HIDDEN), idx)
    return out.reshape(B, L, HIDDEN)


# speedup vs baseline: 20.2786x; 1.7544x over previous
import functools

import jax
import jax.numpy as jnp
from jax import lax
from jax.experimental import pallas as pl
from jax.experimental.pallas import tpu as pltpu
from jax.experimental.pallas import tpu_sc as plsc

HIDDEN = 128
NC, NS = 2, 16
NW = NC * NS
GK = 2
GROUP = GK * 128
CHUNK = 8
SREP = 16


def _fuse_body(emb_ref, w_ref, b_ref, out_ref):
    fused = lax.dot_general(
        emb_ref[...], w_ref[...],
        dimension_numbers=(((1,), (1,)), ((), ())),
        preferred_element_type=jnp.float32,
    ) + b_ref[...]
    out_ref[...] = jnp.broadcast_to(fused[None], (SREP, HIDDEN, HIDDEN))


def _fuse_table(emb_pad, W, b2):
    return pl.pallas_call(
        _fuse_body,
        out_shape=jax.ShapeDtypeStruct((SREP, HIDDEN, HIDDEN), jnp.float32),
    )(emb_pad, W, b2)


def _make_gather(n_tokens):
    per_w = n_tokens // NW
    groups = per_w // GROUP
    supers = groups // CHUNK
    idx_rows_per_w = per_w // 128
    mesh = plsc.VectorSubcoreMesh(core_axis_name="c", subcore_axis_name="s")

    @functools.partial(
        pl.kernel,
        mesh=mesh,
        out_type=jax.ShapeDtypeStruct((n_tokens, HIDDEN), jnp.float32),
        scratch_types=[
            pltpu.VMEM_SHARED((SREP * HIDDEN, HIDDEN), jnp.float32),
            pltpu.VMEM((CHUNK * GK, 128), jnp.int32),
            pltpu.VMEM((GROUP, HIDDEN), jnp.float32),
            pltpu.VMEM((GROUP, HIDDEN), jnp.float32),
            pltpu.SemaphoreType.DMA,
            pltpu.SemaphoreType.DMA,
            pltpu.SemaphoreType.DMA,
            pltpu.SemaphoreType.DMA,
        ],
    )
    def gather(table_hbm, idx_hbm, out_hbm, stab, idx_c, rows0, rows1,
               gsem0, gsem1, ssem0, ssem1):
        sid = lax.axis_index("s")
        wid = sid * NC + lax.axis_index("c")
        row_base = wid * per_w
        idx_base = wid * idx_rows_per_w
        rows = (rows0, rows1)
        gsems = (gsem0, gsem1)
        ssems = (ssem0, ssem1)

        def load_chunk(ci):
            pltpu.sync_copy(
                idx_hbm.at[pl.ds(idx_base + ci * (CHUNK * GK), CHUNK * GK)],
                idx_c)
            lane = lax.iota(jnp.int32, 16) + wid
            for r in range(CHUNK * GK):
                off = ((lane + r) % SREP) * HIDDEN
                for c in range(HIDDEN // 16):
                    sl = (r, pl.ds(c * 16, 16))
                    idx_c[sl] = idx_c[sl] + off

        def fire_gather(slot, p):
            for j2 in range(GK):
                pltpu.async_copy(
                    stab.at[idx_c.at[slot * GK + j2]],
                    rows[p].at[pl.ds(j2 * 128, 128)],
                    gsems[p])

        def wait_gather(p):
            for j2 in range(GK):
                pltpu.make_async_copy(
                    stab.at[idx_c.at[j2]],
                    rows[p].at[pl.ds(j2 * 128, 128)],
                    gsems[p]).wait()

        def fire_scatter(g, p):
            pltpu.async_copy(
                rows[p], out_hbm.at[pl.ds(row_base + g * GROUP, GROUP)],
                ssems[p])

        def wait_scatter(q):
            pltpu.make_async_copy(
                rows[q], out_hbm.at[pl.ds(row_base, GROUP)], ssems[q]).wait()

        def super_body(si, first=False, last=False):
            for j in range(CHUNK):
                g = si * CHUNK + j
                p = j % 2
                q = 1 - p
                wait_gather(p)
                fire_scatter(g, p)
                if not (first and j == 0):
                    wait_scatter(q)
                if not (last and j == CHUNK - 1):
                    if j == CHUNK - 1:
                        load_chunk(si + 1)
                        fire_gather(0, q)
                    else:
                        fire_gather(j + 1, q)

        @pl.when(sid == 0)
        def _():
            pltpu.sync_copy(table_hbm, stab)
        plsc.subcore_barrier()

        load_chunk(0)
        fire_gather(0, 0)
        super_body(0, first=True)
        lax.fori_loop(1, supers - 1,
                      lambda si, c: (super_body(si), c)[1], 0)
        super_body(supers - 1, last=True)
        wait_scatter((CHUNK - 1) % 2)

    return gather


def kernel(x, emb, W, b):
    B, L = x.shape
    n = B * L
    emb_pad = jnp.pad(emb, ((0, HIDDEN - emb.shape[0]), (0, 0)))
    fused = _fuse_table(emb_pad, W, b.reshape(1, HIDDEN))
    idx = x.reshape(n // 128, 128)
    out = _make_gather(n)(fused.reshape(SREP * HIDDEN, HIDDEN), idx)
    return out.reshape(B, L, HIDDEN)
